# baseline (speedup 1.0000x reference)
import functools

import jax
import jax.numpy as jnp
from jax import lax
from jax.experimental import pallas as pl
from jax.experimental.pallas import tpu as pltpu
from jax.experimental.pallas import tpu_sc as plsc

N = 10000
E = 320000
D = 128
H = 128
ED = 16
G = 64

NC = 2
NS = 16
NW = NC * NS
EPW = E // NW
CB = 100
NB = EPW // CB
ZPS = N // NS
NPAD = 10112
RPS = NPAD // NS
LANES = 16

_mesh = plsc.VectorSubcoreMesh(core_axis_name="c", subcore_axis_name="s")


@functools.partial(
    pl.kernel,
    out_type=jax.ShapeDtypeStruct((NC, NPAD, D), jnp.float32),
    mesh=_mesh,
    scratch_types=[
        pltpu.VMEM((1, CB), jnp.int32),
        pltpu.VMEM((1, CB), jnp.int32),
        pltpu.VMEM((1, CB), jnp.int32),
        pltpu.VMEM((1, CB), jnp.int32),
        pltpu.VMEM((CB, D), jnp.float32),
        pltpu.VMEM((CB, D), jnp.float32),
        pltpu.VMEM((CB, D), jnp.float32),
        pltpu.VMEM_SHARED((NPAD, D), jnp.float32),
        pltpu.SemaphoreType.DMA,
        pltpu.SemaphoreType.DMA,
        pltpu.SemaphoreType.DMA,
        pltpu.SemaphoreType.DMA,
        pltpu.SemaphoreType.DMA,
        pltpu.SemaphoreType.DMA,
        pltpu.SemaphoreType.DMA,
    ],
)
def _sc_propagate(h_hbm, e_hbm, src_hbm, dst_hbm, part_hbm,
                  si0, si1, di0, di1, rows_v, m0, m1, agg_sh,
                  sg, se0, se1, ssi0, ssi1, sdi0, sdi1):
    c = lax.axis_index("c")
    s = lax.axis_index("s")
    wid = s * NC + c
    msgs = (m0, m1)
    sidx = (si0, si1)
    didx = (di0, di1)
    ses = (se0, se1)
    ssi = (ssi0, ssi1)
    sdi = (sdi0, sdi1)

    @plsc.parallel_loop(0, CB, 1, unroll=4)
    def _(i):
        for q in range(D // LANES):
            m0[i, pl.ds(q * LANES, LANES)] = jnp.zeros((LANES,), jnp.float32)
    for k in range(6):
        pltpu.sync_copy(m0, agg_sh.at[pl.ds(s * ZPS + k * CB, CB)])
    pltpu.sync_copy(m0.at[pl.ds(0, 25)],
                    agg_sh.at[pl.ds(s * ZPS + 6 * CB, 25)])
    plsc.subcore_barrier()

    pltpu.async_copy(src_hbm.at[wid, 0], sidx[0], ssi[0])
    pltpu.async_copy(dst_hbm.at[wid, 0], didx[0], sdi[0])
    pltpu.async_copy(src_hbm.at[wid, 1], sidx[1], ssi[1])
    pltpu.async_copy(dst_hbm.at[wid, 1], didx[1], sdi[1])
    pltpu.make_async_copy(src_hbm.at[wid, 0], sidx[0], ssi[0]).wait()
    pltpu.async_copy(h_hbm.at[sidx[0].at[0]], rows_v, sg)
    pltpu.async_copy(e_hbm.at[wid, 0], msgs[0], ses[0])

    def pair(jj, carry):
        for b in range(2):
            j = 2 * jj + b
            o = 1 - b
            not_last = jj < NB // 2 - 1
            pltpu.make_async_copy(h_hbm.at[sidx[b].at[0]], rows_v, sg).wait()

            @pl.when(not_last)
            def _():
                pltpu.async_copy(src_hbm.at[wid, j + 2], sidx[b], ssi[b])

            def launch_e():
                pltpu.async_copy(e_hbm.at[wid, j + 1], msgs[o], ses[o])

            if b == 0:
                launch_e()
            else:
                pl.when(not_last)(launch_e)

            pltpu.make_async_copy(e_hbm.at[wid, j], msgs[b], ses[b]).wait()

            mb = msgs[b]

            @plsc.parallel_loop(0, CB, 1, unroll=4)
            def _(i):
                for q in range(D // LANES):
                    sl = pl.ds(q * LANES, LANES)
                    mb[i, sl] = jnp.maximum(mb[i, sl] + rows_v[i, sl], 0.0)

            def launch_gather():
                pltpu.make_async_copy(src_hbm.at[wid, j + 1], sidx[o], ssi[o]).wait()
                pltpu.async_copy(h_hbm.at[sidx[o].at[0]], rows_v, sg)

            if b == 0:
                launch_gather()
            else:
                pl.when(not_last)(launch_gather)

            pltpu.make_async_copy(dst_hbm.at[wid, j], didx[b], sdi[b]).wait()
            pltpu.sync_copy(mb, agg_sh.at[didx[b].at[0]], add=True)

            @pl.when(not_last)
            def _():
                pltpu.async_copy(dst_hbm.at[wid, j + 2], didx[b], sdi[b])
        return carry
    lax.fori_loop(0, NB // 2, pair, 0)

    plsc.subcore_barrier()
    pltpu.sync_copy(agg_sh.at[pl.ds(s * RPS, RPS)],
                    part_hbm.at[c, pl.ds(s * RPS, RPS)])


EB = 4000


def _edge_body(ea_ref, We1_ref, be1_ref, We2_ref, be2_ref, e1_ref, e2_ref):
    ea = ea_ref[...]
    e1_ref[...] = jnp.dot(ea, We1_ref[...],
                          preferred_element_type=jnp.float32) + be1_ref[0]
    e2_ref[...] = jnp.dot(ea, We2_ref[...],
                          preferred_element_type=jnp.float32) + be2_ref[0]


def _edge_mlp(edge_attr, We1, be1, We2, be2):
    nblk = E // EB
    return pl.pallas_call(
        _edge_body,
        grid=(nblk,),
        in_specs=[
            pl.BlockSpec((EB, ED), lambda i: (i, 0)),
            pl.BlockSpec((ED, D), lambda i: (0, 0)),
            pl.BlockSpec((1, D), lambda i: (0, 0)),
            pl.BlockSpec((ED, H), lambda i: (0, 0)),
            pl.BlockSpec((1, H), lambda i: (0, 0)),
        ],
        out_specs=[
            pl.BlockSpec((EB, D), lambda i: (i, 0)),
            pl.BlockSpec((EB, H), lambda i: (i, 0)),
        ],
        out_shape=[
            jax.ShapeDtypeStruct((E, D), jnp.float32),
            jax.ShapeDtypeStruct((E, H), jnp.float32),
        ],
    )(edge_attr, We1, be1.reshape(1, D), We2, be2.reshape(1, H))


R = 400
NRB = N // R


def _mlp_body(part_ref, h_ref, scale_ref, Wa_ref, ba_ref, ga_ref, bba_ref,
              Wb_ref, bb_ref, gb_ref, bbb_ref, batch_ref, x_ref, pool_ref):
    i = pl.program_id(0)
    agg = part_ref[0] + part_ref[1] + scale_ref[0] * h_ref[...]
    t = jnp.dot(agg, Wa_ref[...], preferred_element_type=jnp.float32) + ba_ref[0]
    t = t * ga_ref[0] + bba_ref[0]
    t = jnp.maximum(t, 0.0)
    u = jnp.dot(t, Wb_ref[...], preferred_element_type=jnp.float32) + bb_ref[0]
    u = jnp.maximum(u, 0.0)
    u = u * gb_ref[0] + bbb_ref[0]
    x_ref[...] = u
    b = batch_ref[0]
    onehot = (lax.broadcasted_iota(jnp.int32, (G, R), 0) == b).astype(jnp.float32)
    ppart = jnp.dot(onehot, u, preferred_element_type=jnp.float32)

    @pl.when(i == 0)
    def _():
        pool_ref[...] = ppart

    @pl.when(i != 0)
    def _():
        pool_ref[...] += ppart


def _node_mlp(part, h, scale, Wa, ba, ga_eff, bba, Wb, bb, gb_eff, bbb, batch3):
    return pl.pallas_call(
        _mlp_body,
        grid=(NRB,),
        in_specs=[
            pl.BlockSpec((NC, R, D), lambda i: (0, i, 0)),
            pl.BlockSpec((R, D), lambda i: (i, 0)),
            pl.BlockSpec((1, D), lambda i: (0, 0)),
            pl.BlockSpec((D, H), lambda i: (0, 0)),
            pl.BlockSpec((1, H), lambda i: (0, 0)),
            pl.BlockSpec((1, H), lambda i: (0, 0)),
            pl.BlockSpec((1, H), lambda i: (0, 0)),
            pl.BlockSpec((H, H), lambda i: (0, 0)),
            pl.BlockSpec((1, H), lambda i: (0, 0)),
            pl.BlockSpec((1, H), lambda i: (0, 0)),
            pl.BlockSpec((1, H), lambda i: (0, 0)),
            pl.BlockSpec((1, 1, R), lambda i: (i, 0, 0)),
        ],
        out_specs=[
            pl.BlockSpec((R, H), lambda i: (i, 0)),
            pl.BlockSpec((G, H), lambda i: (0, 0)),
        ],
        out_shape=[
            jax.ShapeDtypeStruct((N, H), jnp.float32),
            jax.ShapeDtypeStruct((G, H), jnp.float32),
        ],
    )(part, h, scale, Wa, ba, ga_eff, bba, Wb, bb, gb_eff, bbb, batch3)


def _head_body(p1_ref, p2_ref, Wl1_ref, bl1_ref, Wl2_ref, bl2_ref, o_ref):
    hcat = jnp.concatenate([p1_ref[...], p2_ref[...]], axis=1)
    t = jnp.dot(hcat, Wl1_ref[...], preferred_element_type=jnp.float32) + bl1_ref[0]
    t = jnp.maximum(t, 0.0)
    o = jnp.dot(t, Wl2_ref[...], preferred_element_type=jnp.float32) + bl2_ref[0]
    o_ref[...] = jax.nn.sigmoid(o)


def _head(p1, p2, Wl1, bl1, Wl2p, bl2p):
    return pl.pallas_call(
        _head_body,
        out_shape=jax.ShapeDtypeStruct((G, H), jnp.float32),
    )(p1, p2, Wl1, bl1, Wl2p, bl2p)


_BN = 1.0 / (1.0 + 1e-5) ** 0.5


def kernel(x, edge_index, edge_attr, batch, We1, be1, eps1, W11, b11, g11,
           bb11, W12, b12, g12, bb12, We2, be2, eps2, W21, b21, g21, bb21,
           W22, b22, g22, bb22, Wl1, bl1, Wl2, bl2):
    src = edge_index[0].reshape(NW, NB, 1, CB)
    dst = edge_index[1].reshape(NW, NB, 1, CB)
    batch3 = batch.reshape(NRB, 1, R)

    e1, e2 = _edge_mlp(edge_attr, We1, be1, We2, be2)
    e1 = e1.reshape(NW, NB, CB, D)
    e2 = e2.reshape(NW, NB, CB, H)

    scale1 = jnp.full((1, D), 1.0, jnp.float32) * (1.0 + eps1)
    scale2 = jnp.full((1, H), 1.0, jnp.float32) * (1.0 + eps2)

    part1 = _sc_propagate(x, e1, src, dst)
    x1, p1 = _node_mlp(part1, x, scale1, W11, b11.reshape(1, H),
                       (g11 * _BN).reshape(1, H), bb11.reshape(1, H),
                       W12, b12.reshape(1, H), (g12 * _BN).reshape(1, H),
                       bb12.reshape(1, H), batch3)

    part2 = _sc_propagate(x1, e2, src, dst)
    x2, p2 = _node_mlp(part2, x1, scale2, W21, b21.reshape(1, H),
                       (g21 * _BN).reshape(1, H), bb21.reshape(1, H),
                       W22, b22.reshape(1, H), (g22 * _BN).reshape(1, H),
                       bb22.reshape(1, H), batch3)

    Wl2p = jnp.pad(Wl2, ((0, 0), (0, H - 1)))
    bl2p = jnp.pad(bl2, (0, H - 1)).reshape(1, H)
    o = _head(p1, p2, Wl1, bl1.reshape(1, 2 * H), Wl2p, bl2p)
    return o[:, :1]


# speedup vs baseline: 1.1881x; 1.1881x over previous
import functools

import jax
import jax.numpy as jnp
from jax import lax
from jax.experimental import pallas as pl
from jax.experimental.pallas import tpu as pltpu
from jax.experimental.pallas import tpu_sc as plsc

N = 10000
E = 320000
D = 128
H = 128
ED = 16
G = 64

NC = 2
NS = 16
NW = NC * NS
EPW = E // NW
CB = 80
NB = EPW // CB
NBP = NB // 2
ZPS = N // NS
NPAD = 10112
RPS = NPAD // NS
LANES = 16

_mesh = plsc.VectorSubcoreMesh(core_axis_name="c", subcore_axis_name="s")


@functools.partial(
    pl.kernel,
    out_type=jax.ShapeDtypeStruct((NC, NPAD, D), jnp.float32),
    mesh=_mesh,
    scratch_types=[
        pltpu.VMEM((1, CB), jnp.int32),
        pltpu.VMEM((1, CB), jnp.int32),
        pltpu.VMEM((1, CB), jnp.int32),
        pltpu.VMEM((1, CB), jnp.int32),
        pltpu.VMEM((CB, D), jnp.float32),
        pltpu.VMEM((CB, D), jnp.float32),
        pltpu.VMEM((CB, D), jnp.float32),
        pltpu.VMEM_SHARED((NPAD, D), jnp.float32),
        pltpu.SemaphoreType.DMA,
        pltpu.SemaphoreType.DMA,
        pltpu.SemaphoreType.DMA,
        pltpu.SemaphoreType.DMA,
        pltpu.SemaphoreType.DMA,
        pltpu.SemaphoreType.DMA,
        pltpu.SemaphoreType.DMA,
    ],
)
def _sc_propagate(h_hbm, e_hbm, src_hbm, dst_hbm, part_hbm,
                  si0, si1, di0, di1, rows_v, m0, m1, agg_sh,
                  sg, se0, se1, ssi0, ssi1, sdi0, sdi1):
    c = lax.axis_index("c")
    s = lax.axis_index("s")
    wid = s * NC + c
    msgs = (m0, m1)
    sidx = (si0, si1)
    didx = (di0, di1)
    ses = (se0, se1)
    ssi = (ssi0, ssi1)
    sdi = (sdi0, sdi1)

    @plsc.parallel_loop(0, CB, 1, unroll=4)
    def _(i):
        for q in range(D // LANES):
            m0[i, pl.ds(q * LANES, LANES)] = jnp.zeros((LANES,), jnp.float32)
    for k in range(7):
        pltpu.sync_copy(m0, agg_sh.at[pl.ds(s * ZPS + k * CB, CB)])
    pltpu.sync_copy(m0.at[pl.ds(0, 65)],
                    agg_sh.at[pl.ds(s * ZPS + 7 * CB, 65)])
    plsc.subcore_barrier()

    pltpu.async_copy(src_hbm.at[wid, 0], sidx[0], ssi[0])
    pltpu.async_copy(dst_hbm.at[wid, 0], didx[0], sdi[0])
    pltpu.async_copy(src_hbm.at[wid, 1], sidx[1], ssi[1])
    pltpu.async_copy(dst_hbm.at[wid, 1], didx[1], sdi[1])
    pltpu.make_async_copy(src_hbm.at[wid, 0], sidx[0], ssi[0]).wait()
    pltpu.async_copy(h_hbm.at[sidx[0].at[0]], rows_v, sg)
    pltpu.async_copy(e_hbm.at[wid, 0], msgs[0], ses[0])

    def body(j, jj, b, not_last2):
        o = 1 - b
        pltpu.make_async_copy(h_hbm.at[sidx[b].at[0]], rows_v, sg).wait()

        def fetch_sidx():
            pltpu.async_copy(src_hbm.at[wid, j + 2], sidx[b], ssi[b])

        if not_last2 is None:
            fetch_sidx()
        else:
            pl.when(not_last2)(fetch_sidx)

        pltpu.async_copy(e_hbm.at[wid, j + 1], msgs[o], ses[o])

        pltpu.make_async_copy(e_hbm.at[wid, j], msgs[b], ses[b]).wait()

        mb = msgs[b]

        @plsc.parallel_loop(0, CB, 1, unroll=4)
        def _(i):
            for q in range(D // LANES):
                sl = pl.ds(q * LANES, LANES)
                mb[i, sl] = jnp.maximum(mb[i, sl] + rows_v[i, sl], 0.0)

        pltpu.make_async_copy(src_hbm.at[wid, j + 1], sidx[o], ssi[o]).wait()
        pltpu.async_copy(h_hbm.at[sidx[o].at[0]], rows_v, sg)

        pltpu.make_async_copy(dst_hbm.at[wid, j], didx[b], sdi[b]).wait()
        pltpu.sync_copy(mb, agg_sh.at[didx[b].at[0]], add=True)

        def fetch_didx():
            pltpu.async_copy(dst_hbm.at[wid, j + 2], didx[b], sdi[b])

        if not_last2 is None:
            fetch_didx()
        else:
            pl.when(not_last2)(fetch_didx)

    def pair(jj, carry):
        body(2 * jj, jj, 0, None)
        body(2 * jj + 1, jj, 1, jj < NBP - 1)
        return carry
    lax.fori_loop(0, NBP, pair, 0)

    jt = NB - 1
    pltpu.make_async_copy(h_hbm.at[sidx[0].at[0]], rows_v, sg).wait()
    pltpu.make_async_copy(e_hbm.at[wid, jt], msgs[0], ses[0]).wait()

    @plsc.parallel_loop(0, CB, 1, unroll=4)
    def _(i):
        for q in range(D // LANES):
            sl = pl.ds(q * LANES, LANES)
            m0[i, sl] = jnp.maximum(m0[i, sl] + rows_v[i, sl], 0.0)

    pltpu.make_async_copy(dst_hbm.at[wid, jt], didx[0], sdi[0]).wait()
    pltpu.sync_copy(m0, agg_sh.at[didx[0].at[0]], add=True)

    plsc.subcore_barrier()
    pltpu.sync_copy(agg_sh.at[pl.ds(s * RPS, RPS)],
                    part_hbm.at[c, pl.ds(s * RPS, RPS)])


EB = 4000


def _edge_body(ea_ref, We1_ref, be1_ref, We2_ref, be2_ref, e1_ref, e2_ref):
    ea = ea_ref[...]
    e1_ref[...] = jnp.dot(ea, We1_ref[...],
                          preferred_element_type=jnp.float32) + be1_ref[0]
    e2_ref[...] = jnp.dot(ea, We2_ref[...],
                          preferred_element_type=jnp.float32) + be2_ref[0]


def _edge_mlp(edge_attr, We1, be1, We2, be2):
    nblk = E // EB
    return pl.pallas_call(
        _edge_body,
        grid=(nblk,),
        in_specs=[
            pl.BlockSpec((EB, ED), lambda i: (i, 0)),
            pl.BlockSpec((ED, D), lambda i: (0, 0)),
            pl.BlockSpec((1, D), lambda i: (0, 0)),
            pl.BlockSpec((ED, H), lambda i: (0, 0)),
            pl.BlockSpec((1, H), lambda i: (0, 0)),
        ],
        out_specs=[
            pl.BlockSpec((EB, D), lambda i: (i, 0)),
            pl.BlockSpec((EB, H), lambda i: (i, 0)),
        ],
        out_shape=[
            jax.ShapeDtypeStruct((E, D), jnp.float32),
            jax.ShapeDtypeStruct((E, H), jnp.float32),
        ],
    )(edge_attr, We1, be1.reshape(1, D), We2, be2.reshape(1, H))


R = 400
NRB = N // R


def _mlp_body(part_ref, h_ref, scale_ref, Wa_ref, ba_ref, ga_ref, bba_ref,
              Wb_ref, bb_ref, gb_ref, bbb_ref, batch_ref, x_ref, pool_ref):
    i = pl.program_id(0)
    agg = part_ref[0] + part_ref[1] + scale_ref[0] * h_ref[...]
    t = jnp.dot(agg, Wa_ref[...], preferred_element_type=jnp.float32) + ba_ref[0]
    t = t * ga_ref[0] + bba_ref[0]
    t = jnp.maximum(t, 0.0)
    u = jnp.dot(t, Wb_ref[...], preferred_element_type=jnp.float32) + bb_ref[0]
    u = jnp.maximum(u, 0.0)
    u = u * gb_ref[0] + bbb_ref[0]
    x_ref[...] = u
    b = batch_ref[0]
    onehot = (lax.broadcasted_iota(jnp.int32, (G, R), 0) == b).astype(jnp.float32)
    ppart = jnp.dot(onehot, u, preferred_element_type=jnp.float32)

    @pl.when(i == 0)
    def _():
        pool_ref[...] = ppart

    @pl.when(i != 0)
    def _():
        pool_ref[...] += ppart


def _node_mlp(part, h, scale, Wa, ba, ga_eff, bba, Wb, bb, gb_eff, bbb, batch3):
    return pl.pallas_call(
        _mlp_body,
        grid=(NRB,),
        in_specs=[
            pl.BlockSpec((NC, R, D), lambda i: (0, i, 0)),
            pl.BlockSpec((R, D), lambda i: (i, 0)),
            pl.BlockSpec((1, D), lambda i: (0, 0)),
            pl.BlockSpec((D, H), lambda i: (0, 0)),
            pl.BlockSpec((1, H), lambda i: (0, 0)),
            pl.BlockSpec((1, H), lambda i: (0, 0)),
            pl.BlockSpec((1, H), lambda i: (0, 0)),
            pl.BlockSpec((H, H), lambda i: (0, 0)),
            pl.BlockSpec((1, H), lambda i: (0, 0)),
            pl.BlockSpec((1, H), lambda i: (0, 0)),
            pl.BlockSpec((1, H), lambda i: (0, 0)),
            pl.BlockSpec((1, 1, R), lambda i: (i, 0, 0)),
        ],
        out_specs=[
            pl.BlockSpec((R, H), lambda i: (i, 0)),
            pl.BlockSpec((G, H), lambda i: (0, 0)),
        ],
        out_shape=[
            jax.ShapeDtypeStruct((N, H), jnp.float32),
            jax.ShapeDtypeStruct((G, H), jnp.float32),
        ],
    )(part, h, scale, Wa, ba, ga_eff, bba, Wb, bb, gb_eff, bbb, batch3)


def _head_body(p1_ref, p2_ref, Wl1_ref, bl1_ref, Wl2_ref, bl2_ref, o_ref):
    hcat = jnp.concatenate([p1_ref[...], p2_ref[...]], axis=1)
    t = jnp.dot(hcat, Wl1_ref[...], preferred_element_type=jnp.float32) + bl1_ref[0]
    t = jnp.maximum(t, 0.0)
    o = jnp.dot(t, Wl2_ref[...], preferred_element_type=jnp.float32) + bl2_ref[0]
    o_ref[...] = jax.nn.sigmoid(o)


def _head(p1, p2, Wl1, bl1, Wl2p, bl2p):
    return pl.pallas_call(
        _head_body,
        out_shape=jax.ShapeDtypeStruct((G, H), jnp.float32),
    )(p1, p2, Wl1, bl1, Wl2p, bl2p)


_BN = 1.0 / (1.0 + 1e-5) ** 0.5


def kernel(x, edge_index, edge_attr, batch, We1, be1, eps1, W11, b11, g11,
           bb11, W12, b12, g12, bb12, We2, be2, eps2, W21, b21, g21, bb21,
           W22, b22, g22, bb22, Wl1, bl1, Wl2, bl2):
    src = edge_index[0].reshape(NW, NB, 1, CB)
    dst = edge_index[1].reshape(NW, NB, 1, CB)
    batch3 = batch.reshape(NRB, 1, R)

    e1, e2 = _edge_mlp(edge_attr, We1, be1, We2, be2)
    e1 = e1.reshape(NW, NB, CB, D)
    e2 = e2.reshape(NW, NB, CB, H)

    scale1 = jnp.full((1, D), 1.0, jnp.float32) * (1.0 + eps1)
    scale2 = jnp.full((1, H), 1.0, jnp.float32) * (1.0 + eps2)

    part1 = _sc_propagate(x, e1, src, dst)
    x1, p1 = _node_mlp(part1, x, scale1, W11, b11.reshape(1, H),
                       (g11 * _BN).reshape(1, H), bb11.reshape(1, H),
                       W12, b12.reshape(1, H), (g12 * _BN).reshape(1, H),
                       bb12.reshape(1, H), batch3)

    part2 = _sc_propagate(x1, e2, src, dst)
    x2, p2 = _node_mlp(part2, x1, scale2, W21, b21.reshape(1, H),
                       (g21 * _BN).reshape(1, H), bb21.reshape(1, H),
                       W22, b22.reshape(1, H), (g22 * _BN).reshape(1, H),
                       bb22.reshape(1, H), batch3)

    Wl2p = jnp.pad(Wl2, ((0, 0), (0, H - 1)))
    bl2p = jnp.pad(bl2, (0, H - 1)).reshape(1, H)
    o = _head(p1, p2, Wl1, bl1.reshape(1, 2 * H), Wl2p, bl2p)
    return o[:, :1]


# speedup vs baseline: 1.4413x; 1.2132x over previous
import functools

import jax
import jax.numpy as jnp
from jax import lax
from jax.experimental import pallas as pl
from jax.experimental.pallas import tpu as pltpu
from jax.experimental.pallas import tpu_sc as plsc

N = 10000
E = 320000
D = 128
H = 128
ED = 16
G = 64

NC = 2
NS = 16
NW = NC * NS
EPW = E // NW
CB = 80
NB = EPW // CB
NBP = NB // 2
ZPS = N // NS
NPAD = 10112
RPS = NPAD // NS
LANES = 16

_mesh = plsc.VectorSubcoreMesh(core_axis_name="c", subcore_axis_name="s")


@functools.partial(
    pl.kernel,
    out_type=jax.ShapeDtypeStruct((NC, NPAD, D), jnp.float32),
    mesh=_mesh,
    scratch_types=[
        pltpu.VMEM((1, CB), jnp.int32),
        pltpu.VMEM((1, CB), jnp.int32),
        pltpu.VMEM((1, CB), jnp.int32),
        pltpu.VMEM((1, CB), jnp.int32),
        pltpu.VMEM((CB, D), jnp.float32),
        pltpu.VMEM((CB, D), jnp.float32),
        pltpu.VMEM((CB, D), jnp.float32),
        pltpu.VMEM_SHARED((NPAD, D), jnp.float32),
        pltpu.SemaphoreType.DMA,
        pltpu.SemaphoreType.DMA,
        pltpu.SemaphoreType.DMA,
        pltpu.SemaphoreType.DMA,
        pltpu.SemaphoreType.DMA,
        pltpu.SemaphoreType.DMA,
        pltpu.SemaphoreType.DMA,
    ],
)
def _sc_propagate(h_hbm, e_hbm, src_hbm, dst_hbm, part_hbm,
                  si0, si1, di0, di1, rows_v, m0, m1, agg_sh,
                  sg, se0, se1, ssi0, ssi1, sdi0, sdi1):
    c = lax.axis_index("c")
    s = lax.axis_index("s")
    wid = s * NC + c
    msgs = (m0, m1)
    sidx = (si0, si1)
    didx = (di0, di1)
    ses = (se0, se1)
    ssi = (ssi0, ssi1)
    sdi = (sdi0, sdi1)

    @plsc.parallel_loop(0, CB, 1, unroll=4)
    def _(i):
        for q in range(D // LANES):
            m0[i, pl.ds(q * LANES, LANES)] = jnp.zeros((LANES,), jnp.float32)
    for k in range(7):
        pltpu.sync_copy(m0, agg_sh.at[pl.ds(s * ZPS + k * CB, CB)])
    pltpu.sync_copy(m0.at[pl.ds(0, 65)],
                    agg_sh.at[pl.ds(s * ZPS + 7 * CB, 65)])
    plsc.subcore_barrier()

    pltpu.async_copy(src_hbm.at[wid, 0], sidx[0], ssi[0])
    pltpu.async_copy(dst_hbm.at[wid, 0], didx[0], sdi[0])
    pltpu.async_copy(src_hbm.at[wid, 1], sidx[1], ssi[1])
    pltpu.async_copy(dst_hbm.at[wid, 1], didx[1], sdi[1])
    pltpu.make_async_copy(src_hbm.at[wid, 0], sidx[0], ssi[0]).wait()
    pltpu.async_copy(h_hbm.at[sidx[0].at[0]], rows_v, sg)
    pltpu.async_copy(e_hbm.at[wid, 0], msgs[0], ses[0])

    def body(j, jj, b, not_last2):
        o = 1 - b
        pltpu.make_async_copy(h_hbm.at[sidx[b].at[0]], rows_v, sg).wait()

        def fetch_sidx():
            pltpu.async_copy(src_hbm.at[wid, j + 2], sidx[b], ssi[b])

        if not_last2 is None:
            fetch_sidx()
        else:
            pl.when(not_last2)(fetch_sidx)

        pltpu.async_copy(e_hbm.at[wid, j + 1], msgs[o], ses[o])

        pltpu.make_async_copy(e_hbm.at[wid, j], msgs[b], ses[b]).wait()

        mb = msgs[b]

        @plsc.parallel_loop(0, CB, 1, unroll=4)
        def _(i):
            for q in range(D // LANES):
                sl = pl.ds(q * LANES, LANES)
                mb[i, sl] = jnp.maximum(mb[i, sl] + rows_v[i, sl], 0.0)

        pltpu.make_async_copy(src_hbm.at[wid, j + 1], sidx[o], ssi[o]).wait()
        pltpu.async_copy(h_hbm.at[sidx[o].at[0]], rows_v, sg)

        pltpu.make_async_copy(dst_hbm.at[wid, j], didx[b], sdi[b]).wait()
        pltpu.sync_copy(mb, agg_sh.at[didx[b].at[0]], add=True)

        def fetch_didx():
            pltpu.async_copy(dst_hbm.at[wid, j + 2], didx[b], sdi[b])

        if not_last2 is None:
            fetch_didx()
        else:
            pl.when(not_last2)(fetch_didx)

    def pair(jj, carry):
        body(2 * jj, jj, 0, None)
        body(2 * jj + 1, jj, 1, jj < NBP - 1)
        return carry
    lax.fori_loop(0, NBP, pair, 0)

    jt = NB - 1
    pltpu.make_async_copy(h_hbm.at[sidx[0].at[0]], rows_v, sg).wait()
    pltpu.make_async_copy(e_hbm.at[wid, jt], msgs[0], ses[0]).wait()

    @plsc.parallel_loop(0, CB, 1, unroll=4)
    def _(i):
        for q in range(D // LANES):
            sl = pl.ds(q * LANES, LANES)
            m0[i, sl] = jnp.maximum(m0[i, sl] + rows_v[i, sl], 0.0)

    pltpu.make_async_copy(dst_hbm.at[wid, jt], didx[0], sdi[0]).wait()
    pltpu.sync_copy(m0, agg_sh.at[didx[0].at[0]], add=True)

    plsc.subcore_barrier()
    pltpu.sync_copy(agg_sh.at[pl.ds(s * RPS, RPS)],
                    part_hbm.at[c, pl.ds(s * RPS, RPS)])


EB = 6400


def _edge_body(eat_ref, We_ref, be_ref, e_ref):
    e_ref[...] = lax.dot_general(
        eat_ref[...], We_ref[...], (((0,), (0,)), ((), ())),
        preferred_element_type=jnp.float32) + be_ref[0]


def _edge_mlp(edge_attr_t, We, be):
    nblk = E // EB
    return pl.pallas_call(
        _edge_body,
        grid=(nblk,),
        in_specs=[
            pl.BlockSpec((ED, EB), lambda i: (0, i)),
            pl.BlockSpec((ED, D), lambda i: (0, 0)),
            pl.BlockSpec((1, D), lambda i: (0, 0)),
        ],
        out_specs=pl.BlockSpec((EB, D), lambda i: (i, 0)),
        out_shape=jax.ShapeDtypeStruct((E, D), jnp.float32),
    )(edge_attr_t, We, be.reshape(1, D))


R = 400
NRB = N // R


def _mlp_body(part_ref, h_ref, scale_ref, Wa_ref, ba_ref, ga_ref, bba_ref,
              Wb_ref, bb_ref, gb_ref, bbb_ref, batch_ref, x_ref, pool_ref):
    i = pl.program_id(0)
    agg = part_ref[0] + part_ref[1] + scale_ref[0] * h_ref[...]
    t = jnp.dot(agg, Wa_ref[...], preferred_element_type=jnp.float32) + ba_ref[0]
    t = t * ga_ref[0] + bba_ref[0]
    t = jnp.maximum(t, 0.0)
    u = jnp.dot(t, Wb_ref[...], preferred_element_type=jnp.float32) + bb_ref[0]
    u = jnp.maximum(u, 0.0)
    u = u * gb_ref[0] + bbb_ref[0]
    x_ref[...] = u
    b = batch_ref[0]
    onehot = (lax.broadcasted_iota(jnp.int32, (G, R), 0) == b).astype(jnp.float32)
    ppart = jnp.dot(onehot, u, preferred_element_type=jnp.float32)

    @pl.when(i == 0)
    def _():
        pool_ref[...] = ppart

    @pl.when(i != 0)
    def _():
        pool_ref[...] += ppart


def _node_mlp(part, h, scale, Wa, ba, ga_eff, bba, Wb, bb, gb_eff, bbb, batch3):
    return pl.pallas_call(
        _mlp_body,
        grid=(NRB,),
        in_specs=[
            pl.BlockSpec((NC, R, D), lambda i: (0, i, 0)),
            pl.BlockSpec((R, D), lambda i: (i, 0)),
            pl.BlockSpec((1, D), lambda i: (0, 0)),
            pl.BlockSpec((D, H), lambda i: (0, 0)),
            pl.BlockSpec((1, H), lambda i: (0, 0)),
            pl.BlockSpec((1, H), lambda i: (0, 0)),
            pl.BlockSpec((1, H), lambda i: (0, 0)),
            pl.BlockSpec((H, H), lambda i: (0, 0)),
            pl.BlockSpec((1, H), lambda i: (0, 0)),
            pl.BlockSpec((1, H), lambda i: (0, 0)),
            pl.BlockSpec((1, H), lambda i: (0, 0)),
            pl.BlockSpec((1, 1, R), lambda i: (i, 0, 0)),
        ],
        out_specs=[
            pl.BlockSpec((R, H), lambda i: (i, 0)),
            pl.BlockSpec((G, H), lambda i: (0, 0)),
        ],
        out_shape=[
            jax.ShapeDtypeStruct((N, H), jnp.float32),
            jax.ShapeDtypeStruct((G, H), jnp.float32),
        ],
    )(part, h, scale, Wa, ba, ga_eff, bba, Wb, bb, gb_eff, bbb, batch3)


def _head_body(p1_ref, p2_ref, Wl1_ref, bl1_ref, Wl2_ref, bl2_ref, o_ref):
    hcat = jnp.concatenate([p1_ref[...], p2_ref[...]], axis=1)
    t = jnp.dot(hcat, Wl1_ref[...], preferred_element_type=jnp.float32) + bl1_ref[0]
    t = jnp.maximum(t, 0.0)
    o = jnp.dot(t, Wl2_ref[...], preferred_element_type=jnp.float32) + bl2_ref[0]
    o_ref[...] = jax.nn.sigmoid(o)


def _head(p1, p2, Wl1, bl1, Wl2p, bl2p):
    return pl.pallas_call(
        _head_body,
        out_shape=jax.ShapeDtypeStruct((G, H), jnp.float32),
    )(p1, p2, Wl1, bl1, Wl2p, bl2p)


_BN = 1.0 / (1.0 + 1e-5) ** 0.5


def kernel(x, edge_index, edge_attr, batch, We1, be1, eps1, W11, b11, g11,
           bb11, W12, b12, g12, bb12, We2, be2, eps2, W21, b21, g21, bb21,
           W22, b22, g22, bb22, Wl1, bl1, Wl2, bl2):
    src = edge_index[0].reshape(NW, NB, 1, CB)
    dst = edge_index[1].reshape(NW, NB, 1, CB)
    batch3 = batch.reshape(NRB, 1, R)

    eat = edge_attr.T
    e1 = _edge_mlp(eat, We1, be1).reshape(NW, NB, CB, D)
    e2 = _edge_mlp(eat, We2, be2).reshape(NW, NB, CB, H)

    scale1 = jnp.full((1, D), 1.0, jnp.float32) * (1.0 + eps1)
    scale2 = jnp.full((1, H), 1.0, jnp.float32) * (1.0 + eps2)

    part1 = _sc_propagate(x, e1, src, dst)
    x1, p1 = _node_mlp(part1, x, scale1, W11, b11.reshape(1, H),
                       (g11 * _BN).reshape(1, H), bb11.reshape(1, H),
                       W12, b12.reshape(1, H), (g12 * _BN).reshape(1, H),
                       bb12.reshape(1, H), batch3)

    part2 = _sc_propagate(x1, e2, src, dst)
    x2, p2 = _node_mlp(part2, x1, scale2, W21, b21.reshape(1, H),
                       (g21 * _BN).reshape(1, H), bb21.reshape(1, H),
                       W22, b22.reshape(1, H), (g22 * _BN).reshape(1, H),
                       bb22.reshape(1, H), batch3)

    Wl2p = jnp.pad(Wl2, ((0, 0), (0, H - 1)))
    bl2p = jnp.pad(bl2, (0, H - 1)).reshape(1, H)
    o = _head(p1, p2, Wl1, bl1.reshape(1, 2 * H), Wl2p, bl2p)
    return o[:, :1]


# speedup vs baseline: 1.5601x; 1.0824x over previous
import functools

import jax
import jax.numpy as jnp
from jax import lax
from jax.experimental import pallas as pl
from jax.experimental.pallas import tpu as pltpu
from jax.experimental.pallas import tpu_sc as plsc

N = 10000
E = 320000
D = 128
H = 128
ED = 16
G = 64

NC = 2
NS = 16
NW = NC * NS
EPW = E // NW
CB = 80
NB = EPW // CB
NBP = NB // 2
ZPS = N // NS
NPAD = 10112
RPS = NPAD // NS
LANES = 16

_mesh = plsc.VectorSubcoreMesh(core_axis_name="c", subcore_axis_name="s")


@functools.partial(
    pl.kernel,
    out_type=jax.ShapeDtypeStruct((NC, NPAD, D), jnp.float32),
    mesh=_mesh,
    scratch_types=[
        pltpu.VMEM((2, CB), jnp.int32),
        pltpu.VMEM((2, CB), jnp.int32),
        pltpu.VMEM((CB, D), jnp.float32),
        pltpu.VMEM((CB, D), jnp.float32),
        pltpu.VMEM((CB, D), jnp.float32),
        pltpu.VMEM((CB, D), jnp.float32),
        pltpu.VMEM_SHARED((NPAD, D), jnp.float32),
        pltpu.SemaphoreType.DMA,
        pltpu.SemaphoreType.DMA,
        pltpu.SemaphoreType.DMA,
        pltpu.SemaphoreType.DMA,
        pltpu.SemaphoreType.DMA,
        pltpu.SemaphoreType.DMA,
    ],
)
def _sc_propagate(h_hbm, e_hbm, idx_hbm, part_hbm,
                  i0, i1, r0, r1, m0, m1, agg_sh,
                  sg0, sg1, se0, se1, si0, si1):
    c = lax.axis_index("c")
    s = lax.axis_index("s")
    wid = s * NC + c
    rows = (r0, r1)
    msgs = (m0, m1)
    idxb = (i0, i1)
    sgs = (sg0, sg1)
    ses = (se0, se1)
    sis = (si0, si1)

    @plsc.parallel_loop(0, CB, 1, unroll=4)
    def _(i):
        for q in range(D // LANES):
            m0[i, pl.ds(q * LANES, LANES)] = jnp.zeros((LANES,), jnp.float32)
    for k in range(7):
        pltpu.sync_copy(m0, agg_sh.at[pl.ds(s * ZPS + k * CB, CB)])
    pltpu.sync_copy(m0.at[pl.ds(0, 65)],
                    agg_sh.at[pl.ds(s * ZPS + 7 * CB, 65)])
    plsc.subcore_barrier()

    pltpu.async_copy(idx_hbm.at[wid, 0], idxb[0], sis[0])
    pltpu.async_copy(idx_hbm.at[wid, 1], idxb[1], sis[1])
    pltpu.make_async_copy(idx_hbm.at[wid, 0], idxb[0], sis[0]).wait()
    pltpu.async_copy(h_hbm.at[idxb[0].at[0]], rows[0], sgs[0])
    pltpu.async_copy(e_hbm.at[wid, 0], msgs[0], ses[0])

    def body(j, b, not_last2):
        o = 1 - b
        pltpu.make_async_copy(h_hbm.at[idxb[b].at[0]], rows[b], sgs[b]).wait()

        pltpu.make_async_copy(idx_hbm.at[wid, j + 1], idxb[o], sis[o]).wait()
        pltpu.async_copy(h_hbm.at[idxb[o].at[0]], rows[o], sgs[o])
        pltpu.async_copy(e_hbm.at[wid, j + 1], msgs[o], ses[o])

        pltpu.make_async_copy(e_hbm.at[wid, j], msgs[b], ses[b]).wait()

        mb = msgs[b]
        rb = rows[b]

        @plsc.parallel_loop(0, CB, 1, unroll=4)
        def _(i):
            for q in range(D // LANES):
                sl = pl.ds(q * LANES, LANES)
                mb[i, sl] = jnp.maximum(mb[i, sl] + rb[i, sl], 0.0)

        pltpu.sync_copy(mb, agg_sh.at[idxb[b].at[1]], add=True)

        def fetch_idx():
            pltpu.async_copy(idx_hbm.at[wid, j + 2], idxb[b], sis[b])

        if not_last2 is None:
            fetch_idx()
        else:
            pl.when(not_last2)(fetch_idx)

    def pair(jj, carry):
        body(2 * jj, 0, None)
        body(2 * jj + 1, 1, jj < NBP - 1)
        return carry
    lax.fori_loop(0, NBP, pair, 0)

    pltpu.make_async_copy(h_hbm.at[idxb[0].at[0]], rows[0], sgs[0]).wait()
    pltpu.make_async_copy(e_hbm.at[wid, NB - 1], msgs[0], ses[0]).wait()

    @plsc.parallel_loop(0, CB, 1, unroll=4)
    def _(i):
        for q in range(D // LANES):
            sl = pl.ds(q * LANES, LANES)
            m0[i, sl] = jnp.maximum(m0[i, sl] + r0[i, sl], 0.0)

    pltpu.sync_copy(m0, agg_sh.at[idxb[0].at[1]], add=True)

    plsc.subcore_barrier()
    pltpu.sync_copy(agg_sh.at[pl.ds(s * RPS, RPS)],
                    part_hbm.at[c, pl.ds(s * RPS, RPS)])


EB = 6400


def _edge_body(eat_ref, We_ref, be_ref, e_ref):
    e_ref[...] = lax.dot_general(
        eat_ref[...], We_ref[...], (((0,), (0,)), ((), ())),
        preferred_element_type=jnp.float32) + be_ref[0]


def _edge_mlp(edge_attr_t, We, be):
    nblk = E // EB
    return pl.pallas_call(
        _edge_body,
        grid=(nblk,),
        in_specs=[
            pl.BlockSpec((ED, EB), lambda i: (0, i)),
            pl.BlockSpec((ED, D), lambda i: (0, 0)),
            pl.BlockSpec((1, D), lambda i: (0, 0)),
        ],
        out_specs=pl.BlockSpec((EB, D), lambda i: (i, 0)),
        out_shape=jax.ShapeDtypeStruct((E, D), jnp.float32),
    )(edge_attr_t, We, be.reshape(1, D))


R = 400
NRB = N // R


def _mlp_body(part_ref, h_ref, scale_ref, Wa_ref, ba_ref, ga_ref, bba_ref,
              Wb_ref, bb_ref, gb_ref, bbb_ref, batch_ref, x_ref, pool_ref):
    i = pl.program_id(0)
    agg = part_ref[0] + part_ref[1] + scale_ref[0] * h_ref[...]
    t = jnp.dot(agg, Wa_ref[...], preferred_element_type=jnp.float32) + ba_ref[0]
    t = t * ga_ref[0] + bba_ref[0]
    t = jnp.maximum(t, 0.0)
    u = jnp.dot(t, Wb_ref[...], preferred_element_type=jnp.float32) + bb_ref[0]
    u = jnp.maximum(u, 0.0)
    u = u * gb_ref[0] + bbb_ref[0]
    x_ref[...] = u
    b = batch_ref[0]
    onehot = (lax.broadcasted_iota(jnp.int32, (G, R), 0) == b).astype(jnp.float32)
    ppart = jnp.dot(onehot, u, preferred_element_type=jnp.float32)

    @pl.when(i == 0)
    def _():
        pool_ref[...] = ppart

    @pl.when(i != 0)
    def _():
        pool_ref[...] += ppart


def _node_mlp(part, h, scale, Wa, ba, ga_eff, bba, Wb, bb, gb_eff, bbb, batch3):
    return pl.pallas_call(
        _mlp_body,
        grid=(NRB,),
        in_specs=[
            pl.BlockSpec((NC, R, D), lambda i: (0, i, 0)),
            pl.BlockSpec((R, D), lambda i: (i, 0)),
            pl.BlockSpec((1, D), lambda i: (0, 0)),
            pl.BlockSpec((D, H), lambda i: (0, 0)),
            pl.BlockSpec((1, H), lambda i: (0, 0)),
            pl.BlockSpec((1, H), lambda i: (0, 0)),
            pl.BlockSpec((1, H), lambda i: (0, 0)),
            pl.BlockSpec((H, H), lambda i: (0, 0)),
            pl.BlockSpec((1, H), lambda i: (0, 0)),
            pl.BlockSpec((1, H), lambda i: (0, 0)),
            pl.BlockSpec((1, H), lambda i: (0, 0)),
            pl.BlockSpec((1, 1, R), lambda i: (i, 0, 0)),
        ],
        out_specs=[
            pl.BlockSpec((R, H), lambda i: (i, 0)),
            pl.BlockSpec((G, H), lambda i: (0, 0)),
        ],
        out_shape=[
            jax.ShapeDtypeStruct((N, H), jnp.float32),
            jax.ShapeDtypeStruct((G, H), jnp.float32),
        ],
    )(part, h, scale, Wa, ba, ga_eff, bba, Wb, bb, gb_eff, bbb, batch3)


def _head_body(p1_ref, p2_ref, Wl1_ref, bl1_ref, Wl2_ref, bl2_ref, o_ref):
    hcat = jnp.concatenate([p1_ref[...], p2_ref[...]], axis=1)
    t = jnp.dot(hcat, Wl1_ref[...], preferred_element_type=jnp.float32) + bl1_ref[0]
    t = jnp.maximum(t, 0.0)
    o = jnp.dot(t, Wl2_ref[...], preferred_element_type=jnp.float32) + bl2_ref[0]
    o_ref[...] = jax.nn.sigmoid(o)


def _head(p1, p2, Wl1, bl1, Wl2p, bl2p):
    return pl.pallas_call(
        _head_body,
        out_shape=jax.ShapeDtypeStruct((G, H), jnp.float32),
    )(p1, p2, Wl1, bl1, Wl2p, bl2p)


_BN = 1.0 / (1.0 + 1e-5) ** 0.5


def kernel(x, edge_index, edge_attr, batch, We1, be1, eps1, W11, b11, g11,
           bb11, W12, b12, g12, bb12, We2, be2, eps2, W21, b21, g21, bb21,
           W22, b22, g22, bb22, Wl1, bl1, Wl2, bl2):
    idx = edge_index.reshape(2, NW, NB, CB).transpose(1, 2, 0, 3)
    batch3 = batch.reshape(NRB, 1, R)

    eat = edge_attr.T
    e1 = _edge_mlp(eat, We1, be1).reshape(NW, NB, CB, D)
    e2 = _edge_mlp(eat, We2, be2).reshape(NW, NB, CB, H)

    scale1 = jnp.full((1, D), 1.0, jnp.float32) * (1.0 + eps1)
    scale2 = jnp.full((1, H), 1.0, jnp.float32) * (1.0 + eps2)

    part1 = _sc_propagate(x, e1, idx)
    x1, p1 = _node_mlp(part1, x, scale1, W11, b11.reshape(1, H),
                       (g11 * _BN).reshape(1, H), bb11.reshape(1, H),
                       W12, b12.reshape(1, H), (g12 * _BN).reshape(1, H),
                       bb12.reshape(1, H), batch3)

    part2 = _sc_propagate(x1, e2, idx)
    x2, p2 = _node_mlp(part2, x1, scale2, W21, b21.reshape(1, H),
                       (g21 * _BN).reshape(1, H), bb21.reshape(1, H),
                       W22, b22.reshape(1, H), (g22 * _BN).reshape(1, H),
                       bb22.reshape(1, H), batch3)

    Wl2p = jnp.pad(Wl2, ((0, 0), (0, H - 1)))
    bl2p = jnp.pad(bl2, (0, H - 1)).reshape(1, H)
    o = _head(p1, p2, Wl1, bl1.reshape(1, 2 * H), Wl2p, bl2p)
    return o[:, :1]


# speedup vs baseline: 1.6405x; 1.0515x over previous
import functools

import jax
import jax.numpy as jnp
from jax import lax
from jax.experimental import pallas as pl
from jax.experimental.pallas import tpu as pltpu
from jax.experimental.pallas import tpu_sc as plsc

N = 10000
E = 320000
D = 128
H = 128
ED = 16
G = 64

NC = 2
NS = 16
NW = NC * NS
EPW = E // NW
CB = 80
NB = EPW // CB
NBP = NB // 2
ZPS = N // NS
NPAD = 10112
RPS = NPAD // NS
LANES = 16

_mesh = plsc.VectorSubcoreMesh(core_axis_name="c", subcore_axis_name="s")


@functools.partial(
    pl.kernel,
    out_type=jax.ShapeDtypeStruct((NC, NPAD, D), jnp.float32),
    mesh=_mesh,
    scratch_types=[
        pltpu.VMEM((2, CB), jnp.int32),
        pltpu.VMEM((2, CB), jnp.int32),
        pltpu.VMEM((CB, D), jnp.float32),
        pltpu.VMEM((CB, D), jnp.float32),
        pltpu.VMEM((CB, D), jnp.float32),
        pltpu.VMEM((CB, D), jnp.float32),
        pltpu.VMEM_SHARED((NPAD, D), jnp.float32),
        pltpu.SemaphoreType.DMA,
        pltpu.SemaphoreType.DMA,
        pltpu.SemaphoreType.DMA,
        pltpu.SemaphoreType.DMA,
        pltpu.SemaphoreType.DMA,
        pltpu.SemaphoreType.DMA,
    ],
)
def _sc_propagate(h_hbm, e_hbm, idx_hbm, part_hbm,
                  i0, i1, r0, r1, m0, m1, agg_sh,
                  sg0, sg1, se0, se1, si0, si1):
    c = lax.axis_index("c")
    s = lax.axis_index("s")
    wid = s * NC + c
    rows = (r0, r1)
    msgs = (m0, m1)
    idxb = (i0, i1)
    sgs = (sg0, sg1)
    ses = (se0, se1)
    sis = (si0, si1)

    @plsc.parallel_loop(0, CB, 1, unroll=4)
    def _(i):
        for q in range(D // LANES):
            m0[i, pl.ds(q * LANES, LANES)] = jnp.zeros((LANES,), jnp.float32)
    for k in range(7):
        pltpu.sync_copy(m0, agg_sh.at[pl.ds(s * ZPS + k * CB, CB)])
    pltpu.sync_copy(m0.at[pl.ds(0, 65)],
                    agg_sh.at[pl.ds(s * ZPS + 7 * CB, 65)])
    plsc.subcore_barrier()

    pltpu.async_copy(idx_hbm.at[wid, 0], idxb[0], sis[0])
    pltpu.async_copy(idx_hbm.at[wid, 1], idxb[1], sis[1])
    pltpu.make_async_copy(idx_hbm.at[wid, 0], idxb[0], sis[0]).wait()
    pltpu.async_copy(h_hbm.at[idxb[0].at[0]], rows[0], sgs[0])
    pltpu.async_copy(e_hbm.at[wid, 0], msgs[0], ses[0])

    def body(j, b, not_last2):
        o = 1 - b
        pltpu.make_async_copy(h_hbm.at[idxb[b].at[0]], rows[b], sgs[b]).wait()

        pltpu.make_async_copy(idx_hbm.at[wid, j + 1], idxb[o], sis[o]).wait()
        pltpu.async_copy(h_hbm.at[idxb[o].at[0]], rows[o], sgs[o])
        pltpu.async_copy(e_hbm.at[wid, j + 1], msgs[o], ses[o])

        pltpu.make_async_copy(e_hbm.at[wid, j], msgs[b], ses[b]).wait()

        mb = msgs[b]
        rb = rows[b]

        @plsc.parallel_loop(0, CB, 1, unroll=4)
        def _(i):
            for q in range(D // LANES):
                sl = pl.ds(q * LANES, LANES)
                mb[i, sl] = jnp.maximum(mb[i, sl] + rb[i, sl], 0.0)

        pltpu.sync_copy(mb, agg_sh.at[idxb[b].at[1]], add=True)

        def fetch_idx():
            pltpu.async_copy(idx_hbm.at[wid, j + 2], idxb[b], sis[b])

        if not_last2 is None:
            fetch_idx()
        else:
            pl.when(not_last2)(fetch_idx)

    def pair(jj, carry):
        body(2 * jj, 0, None)
        body(2 * jj + 1, 1, jj < NBP - 1)
        return carry
    lax.fori_loop(0, NBP, pair, 0)

    pltpu.make_async_copy(h_hbm.at[idxb[0].at[0]], rows[0], sgs[0]).wait()
    pltpu.make_async_copy(e_hbm.at[wid, NB - 1], msgs[0], ses[0]).wait()

    @plsc.parallel_loop(0, CB, 1, unroll=4)
    def _(i):
        for q in range(D // LANES):
            sl = pl.ds(q * LANES, LANES)
            m0[i, sl] = jnp.maximum(m0[i, sl] + r0[i, sl], 0.0)

    pltpu.sync_copy(m0, agg_sh.at[idxb[0].at[1]], add=True)

    plsc.subcore_barrier()
    pltpu.sync_copy(agg_sh.at[pl.ds(s * RPS, RPS)],
                    part_hbm.at[c, pl.ds(s * RPS, RPS)])


EB = 12800


def _edge_body(eat_ref, We_ref, be_ref, e_ref):
    e_ref[...] = lax.dot_general(
        eat_ref[...], We_ref[...], (((0,), (0,)), ((), ())),
        preferred_element_type=jnp.float32) + be_ref[0]


def _edge_mlp(edge_attr_t, We, be):
    nblk = E // EB
    return pl.pallas_call(
        _edge_body,
        grid=(nblk,),
        in_specs=[
            pl.BlockSpec((ED, EB), lambda i: (0, i)),
            pl.BlockSpec((ED, D), lambda i: (0, 0)),
            pl.BlockSpec((1, D), lambda i: (0, 0)),
        ],
        out_specs=pl.BlockSpec((EB, D), lambda i: (i, 0)),
        out_shape=jax.ShapeDtypeStruct((E, D), jnp.float32),
    )(edge_attr_t, We, be.reshape(1, D))


R = 1000
NRB = N // R


def _mlp_body(part_ref, h_ref, scale_ref, Wa_ref, ba_ref, ga_ref, bba_ref,
              Wb_ref, bb_ref, gb_ref, bbb_ref, batch_ref, x_ref, pool_ref):
    i = pl.program_id(0)
    agg = part_ref[0] + part_ref[1] + scale_ref[0] * h_ref[...]
    t = jnp.dot(agg, Wa_ref[...], preferred_element_type=jnp.float32) + ba_ref[0]
    t = t * ga_ref[0] + bba_ref[0]
    t = jnp.maximum(t, 0.0)
    u = jnp.dot(t, Wb_ref[...], preferred_element_type=jnp.float32) + bb_ref[0]
    u = jnp.maximum(u, 0.0)
    u = u * gb_ref[0] + bbb_ref[0]
    x_ref[...] = u
    b = batch_ref[0]
    onehot = (lax.broadcasted_iota(jnp.int32, (G, R), 0) == b).astype(jnp.float32)
    ppart = jnp.dot(onehot, u, preferred_element_type=jnp.float32)

    @pl.when(i == 0)
    def _():
        pool_ref[...] = ppart

    @pl.when(i != 0)
    def _():
        pool_ref[...] += ppart


def _node_mlp(part, h, scale, Wa, ba, ga_eff, bba, Wb, bb, gb_eff, bbb, batch3):
    return pl.pallas_call(
        _mlp_body,
        grid=(NRB,),
        in_specs=[
            pl.BlockSpec((NC, R, D), lambda i: (0, i, 0)),
            pl.BlockSpec((R, D), lambda i: (i, 0)),
            pl.BlockSpec((1, D), lambda i: (0, 0)),
            pl.BlockSpec((D, H), lambda i: (0, 0)),
            pl.BlockSpec((1, H), lambda i: (0, 0)),
            pl.BlockSpec((1, H), lambda i: (0, 0)),
            pl.BlockSpec((1, H), lambda i: (0, 0)),
            pl.BlockSpec((H, H), lambda i: (0, 0)),
            pl.BlockSpec((1, H), lambda i: (0, 0)),
            pl.BlockSpec((1, H), lambda i: (0, 0)),
            pl.BlockSpec((1, H), lambda i: (0, 0)),
            pl.BlockSpec((1, 1, R), lambda i: (i, 0, 0)),
        ],
        out_specs=[
            pl.BlockSpec((R, H), lambda i: (i, 0)),
            pl.BlockSpec((G, H), lambda i: (0, 0)),
        ],
        out_shape=[
            jax.ShapeDtypeStruct((N, H), jnp.float32),
            jax.ShapeDtypeStruct((G, H), jnp.float32),
        ],
    )(part, h, scale, Wa, ba, ga_eff, bba, Wb, bb, gb_eff, bbb, batch3)


def _head_body(p1_ref, p2_ref, Wl1_ref, bl1_ref, Wl2_ref, bl2_ref, o_ref):
    hcat = jnp.concatenate([p1_ref[...], p2_ref[...]], axis=1)
    t = jnp.dot(hcat, Wl1_ref[...], preferred_element_type=jnp.float32) + bl1_ref[0]
    t = jnp.maximum(t, 0.0)
    o = jnp.dot(t, Wl2_ref[...], preferred_element_type=jnp.float32) + bl2_ref[0]
    o_ref[...] = jax.nn.sigmoid(o)


def _head(p1, p2, Wl1, bl1, Wl2p, bl2p):
    return pl.pallas_call(
        _head_body,
        out_shape=jax.ShapeDtypeStruct((G, H), jnp.float32),
    )(p1, p2, Wl1, bl1, Wl2p, bl2p)


_BN = 1.0 / (1.0 + 1e-5) ** 0.5


def kernel(x, edge_index, edge_attr, batch, We1, be1, eps1, W11, b11, g11,
           bb11, W12, b12, g12, bb12, We2, be2, eps2, W21, b21, g21, bb21,
           W22, b22, g22, bb22, Wl1, bl1, Wl2, bl2):
    idx = edge_index.reshape(2, NW, NB, CB).transpose(1, 2, 0, 3)
    batch3 = batch.reshape(NRB, 1, R)

    eat = edge_attr.T
    e1 = _edge_mlp(eat, We1, be1).reshape(NW, NB, CB, D)
    e2 = _edge_mlp(eat, We2, be2).reshape(NW, NB, CB, H)

    scale1 = jnp.full((1, D), 1.0, jnp.float32) * (1.0 + eps1)
    scale2 = jnp.full((1, H), 1.0, jnp.float32) * (1.0 + eps2)

    part1 = _sc_propagate(x, e1, idx)
    x1, p1 = _node_mlp(part1, x, scale1, W11, b11.reshape(1, H),
                       (g11 * _BN).reshape(1, H), bb11.reshape(1, H),
                       W12, b12.reshape(1, H), (g12 * _BN).reshape(1, H),
                       bb12.reshape(1, H), batch3)

    part2 = _sc_propagate(x1, e2, idx)
    x2, p2 = _node_mlp(part2, x1, scale2, W21, b21.reshape(1, H),
                       (g21 * _BN).reshape(1, H), bb21.reshape(1, H),
                       W22, b22.reshape(1, H), (g22 * _BN).reshape(1, H),
                       bb22.reshape(1, H), batch3)

    Wl2p = jnp.pad(Wl2, ((0, 0), (0, H - 1)))
    bl2p = jnp.pad(bl2, (0, H - 1)).reshape(1, H)
    o = _head(p1, p2, Wl1, bl1.reshape(1, 2 * H), Wl2p, bl2p)
    return o[:, :1]
